# baseline (speedup 1.0000x reference)
import functools

import jax
import jax.numpy as jnp
from jax import lax
from jax.experimental import pallas as pl
from jax.experimental.pallas import tpu as pltpu
from jax.experimental.pallas import tpu_sc as plsc

B, T, H, V = 1024, 200, 64, 100000
N_STATIC, N_KC, N_KR, N_OBS = 3, 2, 4, 3
NK = N_KR + N_KC

NW = 32
R_KC = B * T * N_KC
KC_PER_W = R_KC // NW
CHUNK = 128
NCH = KC_PER_W // CHUNK
R_ST = B * N_STATIC
ST_PER_W = R_ST // NW


def _sc_gather(tkc, ikc, tst, ist):
    mesh = plsc.VectorSubcoreMesh(core_axis_name="c", subcore_axis_name="s")

    @functools.partial(
        pl.kernel,
        mesh=mesh,
        out_type=(
            jax.ShapeDtypeStruct((R_KC, H), jnp.float32),
            jax.ShapeDtypeStruct((R_ST, H), jnp.float32),
        ),
        scratch_types=[
            pltpu.VMEM((NCH, CHUNK), jnp.int32),
            pltpu.VMEM((2, CHUNK, H), jnp.float32),
            pltpu.VMEM((ST_PER_W,), jnp.int32),
            pltpu.VMEM((ST_PER_W, H), jnp.float32),
            pltpu.SemaphoreType.DMA,
            pltpu.SemaphoreType.DMA,
        ],
        compiler_params=pltpu.CompilerParams(use_tc_tiling_on_sc=False),
    )
    def k(tkc, ikc, tst, ist, out_kc, out_st, idx_v, buf, sidx_v, sbuf, sem0, sem1):
        wid = lax.axis_index("s") * 2 + lax.axis_index("c")
        base = wid * KC_PER_W
        pltpu.sync_copy(ist.at[wid], sidx_v)
        pltpu.async_copy(tst.at[sidx_v], sbuf, sem0).wait()
        pltpu.sync_copy(sbuf, out_st.at[pl.ds(wid * ST_PER_W, ST_PER_W)])
        pltpu.sync_copy(ikc.at[wid], idx_v)
        pltpu.async_copy(tkc.at[idx_v.at[0]], buf.at[0], sem0)
        pltpu.async_copy(tkc.at[idx_v.at[1]], buf.at[1], sem1)

        def body(g, carry):
            c0 = 2 * g
            c1 = c0 + 1
            pltpu.make_async_copy(tkc.at[idx_v.at[c0]], buf.at[0], sem0).wait()
            pltpu.sync_copy(buf.at[0], out_kc.at[pl.ds(base + c0 * CHUNK, CHUNK)])

            @pl.when(c0 + 2 < NCH)
            def _():
                pltpu.async_copy(tkc.at[idx_v.at[c0 + 2]], buf.at[0], sem0)

            pltpu.make_async_copy(tkc.at[idx_v.at[c1]], buf.at[1], sem1).wait()
            pltpu.sync_copy(buf.at[1], out_kc.at[pl.ds(base + c1 * CHUNK, CHUNK)])

            @pl.when(c1 + 2 < NCH)
            def _():
                pltpu.async_copy(tkc.at[idx_v.at[c1 + 2]], buf.at[1], sem1)

            return carry

        lax.fori_loop(0, NCH // 2, body, 0)

    return k(tkc, ikc, tst, ist)


RBLK = 2048
NBLK = (B * T) // RBLK


def _tc_assemble(x_kn, e, x_obs, M_kr, S_e, b_kn, M_obs, b_ob):
    def body(x_ref, e_ref, o_ref, mkr, se, bkn, mobs, bob, kn_out, obs_out):
        kn = jnp.dot(e_ref[...], se[...], preferred_element_type=jnp.float32)
        kn = kn + jnp.dot(x_ref[...], mkr[...], preferred_element_type=jnp.float32)
        kn_out[...] = kn + bkn[...]
        ob = jnp.dot(o_ref[...], mobs[...], preferred_element_type=jnp.float32)
        obs_out[...] = ob + bob[...]

    return pl.pallas_call(
        body,
        grid=(NBLK,),
        in_specs=[
            pl.BlockSpec((RBLK, N_KR), lambda i: (i, 0)),
            pl.BlockSpec((RBLK, N_KC * H), lambda i: (i, 0)),
            pl.BlockSpec((RBLK, N_OBS), lambda i: (i, 0)),
            pl.BlockSpec((N_KR, NK * H), lambda i: (0, 0)),
            pl.BlockSpec((N_KC * H, NK * H), lambda i: (0, 0)),
            pl.BlockSpec((1, NK * H), lambda i: (0, 0)),
            pl.BlockSpec((N_OBS, N_OBS * H), lambda i: (0, 0)),
            pl.BlockSpec((1, N_OBS * H), lambda i: (0, 0)),
        ],
        out_specs=[
            pl.BlockSpec((RBLK, NK * H), lambda i: (i, 0)),
            pl.BlockSpec((RBLK, N_OBS * H), lambda i: (i, 0)),
        ],
        out_shape=[
            jax.ShapeDtypeStruct((B * T, NK * H), jnp.float32),
            jax.ShapeDtypeStruct((B * T, N_OBS * H), jnp.float32),
        ],
    )(x_kn, e, x_obs, M_kr, S_e, b_kn, M_obs, b_ob)


def kernel(static, known_real, known_categorical, observed, E_static, E_kc,
           W_kr, b_kr, W_obs, b_obs):
    f32 = jnp.float32
    kc = known_categorical.astype(jnp.int32).reshape(B * T, N_KC)
    idx_kc = (kc + (jnp.arange(N_KC, dtype=jnp.int32) * V)[None, :]).reshape(
        NW, NCH, CHUNK)
    idx_st = (static.astype(jnp.int32)
              + (jnp.arange(N_STATIC, dtype=jnp.int32) * V)[None, :]).reshape(
        NW, ST_PER_W)
    tkc = E_kc.reshape(N_KC * V, H)
    tst = E_static.reshape(N_STATIC * V, H)

    egather, static_rows = _sc_gather(tkc, idx_kc, tst, idx_st)
    static_emb = static_rows.reshape(B, N_STATIC, H)

    mkr_mask = (jnp.arange(NK)[None, :] == jnp.arange(N_KR)[:, None]).astype(f32)
    M_kr = (W_kr[:, :, None] * mkr_mask[:, None, :]).reshape(N_KR, H * NK)
    b_kn = jnp.einsum("jh,ji->hi", b_kr, mkr_mask).reshape(1, H * NK)
    eyeH = jnp.eye(H, dtype=f32)
    sel = (jnp.arange(NK)[None, :] == (N_KR + jnp.arange(N_KC))[:, None]).astype(f32)
    S_e = (eyeH[None, :, :, None] * sel[:, None, None, :]).reshape(N_KC * H, H * NK)
    eyeO = jnp.eye(N_OBS, dtype=f32)
    M_obs = (W_obs[:, :, None] * eyeO[:, None, :]).reshape(N_OBS, H * N_OBS)
    b_ob = jnp.einsum("jh,ji->hi", b_obs, eyeO).reshape(1, H * N_OBS)

    x_kn = known_real.reshape(B * T, N_KR)
    x_obs = observed.reshape(B * T, N_OBS)
    e = egather.reshape(B * T, N_KC * H)
    kn_flat, obs_flat = _tc_assemble(x_kn, e, x_obs, M_kr, S_e, b_kn, M_obs, b_ob)
    return (static_emb,
            kn_flat.reshape(B, T, H, NK),
            obs_flat.reshape(B, T, H, N_OBS))


# speedup vs baseline: 4.2404x; 4.2404x over previous
import functools

import jax
import jax.numpy as jnp
from jax import lax
from jax.experimental import pallas as pl
from jax.experimental.pallas import tpu as pltpu
from jax.experimental.pallas import tpu_sc as plsc

B, T, H, V = 1024, 200, 64, 100000
N_STATIC, N_KC, N_KR, N_OBS = 3, 2, 4, 3
NK = N_KR + N_KC

NW = 32
HPW = (N_KC * H) // NW
SPW = (N_STATIC * H) // NW
TCH = 8
NTCH = T // TCH


def _sc_gather(tkc_t, kc_j, tst_t, st_k):
    mesh = plsc.VectorSubcoreMesh(core_axis_name="c", subcore_axis_name="s")

    @functools.partial(
        pl.kernel,
        mesh=mesh,
        out_type=(
            jax.ShapeDtypeStruct((T, N_KC, H, B), jnp.float32),
            jax.ShapeDtypeStruct((N_STATIC, H, B), jnp.float32),
        ),
        scratch_types=[
            pltpu.VMEM((V,), jnp.float32),
            pltpu.VMEM((TCH, B), jnp.int32),
            pltpu.VMEM((TCH, B), jnp.float32),
        ],
        compiler_params=pltpu.CompilerParams(
            use_tc_tiling_on_sc=True, needs_layout_passes=False),
    )
    def k(tkc, ikc, tst, ist, e_out, st_out, rowbuf, idxbuf, outbuf):
        wid = lax.axis_index("s") * 2 + lax.axis_index("c")

        def gather_groups(nrows):
            def row_body(r, carry):
                for q in range(B // 16):
                    iv = idxbuf[r, pl.ds(16 * q, 16)]
                    ov = plsc.load_gather(rowbuf, [iv])
                    outbuf[r, pl.ds(16 * q, 16)] = ov
                return carry
            lax.fori_loop(0, nrows, row_body, 0)

        j = wid // 16
        hbase = 4 * (wid % 16)
        for m in range(HPW):
            h = hbase + m
            pltpu.sync_copy(tkc.at[j, h], rowbuf)

            def t_body(c, carry):
                t0 = c * TCH
                pltpu.sync_copy(ikc.at[j, pl.ds(t0, TCH)], idxbuf)
                gather_groups(TCH)
                pltpu.sync_copy(outbuf, e_out.at[pl.ds(t0, TCH), j, h])
                return carry

            lax.fori_loop(0, NTCH, t_body, 0)

        for m in range(SPW):
            p = wid * SPW + m
            kk = p // H
            h = p % H
            pltpu.sync_copy(tst.at[kk, h], rowbuf)
            pltpu.sync_copy(ist.at[kk], idxbuf.at[0])
            gather_groups(1)
            pltpu.sync_copy(outbuf.at[0], st_out.at[kk, h])

    return k(tkc_t, kc_j, tst_t, st_k)


def _tc_assemble(xr_t, obs_t, e, W_kr, b_kr, W_obs, b_obs):
    def body(x_ref, o_ref, e_ref, wk, bk, wo, bo, kn_out, ob_out):
        x = x_ref[0]
        wkv = wk[...][:, :, None]
        bkv = bk[...][:, :, None]
        kn_out[0, 0:N_KR] = x[:, None, :] * wkv + bkv
        kn_out[0, N_KR:NK] = e_ref[0]
        o = o_ref[0]
        ob_out[0] = o[:, None, :] * wo[...][:, :, None] + bo[...][:, :, None]

    return pl.pallas_call(
        body,
        grid=(T,),
        in_specs=[
            pl.BlockSpec((1, N_KR, B), lambda i: (i, 0, 0)),
            pl.BlockSpec((1, N_OBS, B), lambda i: (i, 0, 0)),
            pl.BlockSpec((1, N_KC, H, B), lambda i: (i, 0, 0, 0)),
            pl.BlockSpec((N_KR, H), lambda i: (0, 0)),
            pl.BlockSpec((N_KR, H), lambda i: (0, 0)),
            pl.BlockSpec((N_OBS, H), lambda i: (0, 0)),
            pl.BlockSpec((N_OBS, H), lambda i: (0, 0)),
        ],
        out_specs=[
            pl.BlockSpec((1, NK, H, B), lambda i: (i, 0, 0, 0)),
            pl.BlockSpec((1, N_OBS, H, B), lambda i: (i, 0, 0, 0)),
        ],
        out_shape=[
            jax.ShapeDtypeStruct((T, NK, H, B), jnp.float32),
            jax.ShapeDtypeStruct((T, N_OBS, H, B), jnp.float32),
        ],
    )(xr_t, obs_t, e, W_kr, b_kr, W_obs, b_obs)


def kernel(static, known_real, known_categorical, observed, E_static, E_kc,
           W_kr, b_kr, W_obs, b_obs):
    tkc_t = jnp.transpose(E_kc, (0, 2, 1))
    tst_t = jnp.transpose(E_static, (0, 2, 1))
    kc_j = jnp.transpose(known_categorical.astype(jnp.int32), (2, 1, 0))
    st_k = jnp.transpose(static.astype(jnp.int32), (1, 0))
    xr_t = jnp.transpose(known_real, (1, 2, 0))
    obs_t = jnp.transpose(observed, (1, 2, 0))

    e, st_g = _sc_gather(tkc_t, kc_j, tst_t, st_k)

    kn_t, ob_t = _tc_assemble(xr_t, obs_t, e, W_kr, b_kr, W_obs, b_obs)

    static_emb = jnp.transpose(st_g, (2, 0, 1))
    known = jnp.transpose(kn_t, (3, 0, 2, 1))
    obs = jnp.transpose(ob_t, (3, 0, 2, 1))
    return (static_emb, known, obs)


# speedup vs baseline: 4.9280x; 1.1622x over previous
import functools

import jax
import jax.numpy as jnp
from jax import lax
from jax.experimental import pallas as pl
from jax.experimental.pallas import tpu as pltpu
from jax.experimental.pallas import tpu_sc as plsc

B, T, H, V = 1024, 200, 64, 100000
N_STATIC, N_KC, N_KR, N_OBS = 3, 2, 4, 3
NK = N_KR + N_KC

NW = 32
HPW = (N_KC * H) // NW
SPW = (N_STATIC * H) // NW
TCH = 4
NTCH = T // TCH


def _sc_gather(tkc_t, kc_j, tst_t, st_k):
    mesh = plsc.VectorSubcoreMesh(core_axis_name="c", subcore_axis_name="s")

    @functools.partial(
        pl.kernel,
        mesh=mesh,
        out_type=(
            jax.ShapeDtypeStruct((T, N_KC, H, B), jnp.float32),
            jax.ShapeDtypeStruct((N_STATIC, H, B), jnp.float32),
        ),
        scratch_types=[
            pltpu.VMEM((V,), jnp.float32),
            pltpu.VMEM((2, TCH, B), jnp.int32),
            pltpu.VMEM((2, TCH, B), jnp.float32),
            pltpu.SemaphoreType.DMA,
            pltpu.SemaphoreType.DMA,
            pltpu.SemaphoreType.DMA,
            pltpu.SemaphoreType.DMA,
        ],
        compiler_params=pltpu.CompilerParams(
            use_tc_tiling_on_sc=True, needs_layout_passes=False),
    )
    def k(tkc, ikc, tst, ist, e_out, st_out, rowbuf, idxbuf, outbuf,
          si0, si1, so0, so1):
        wid = lax.axis_index("s") * 2 + lax.axis_index("c")

        def gather_rows(b, nrows):
            def row_body(r, carry):
                for q in range(B // 16):
                    iv = idxbuf[b, r, pl.ds(16 * q, 16)]
                    ov = plsc.load_gather(rowbuf, [iv])
                    outbuf[b, r, pl.ds(16 * q, 16)] = ov
                return carry
            lax.fori_loop(0, nrows, row_body, 0, unroll=True)

        j = wid // 16
        hbase = 4 * (wid % 16)
        for m in range(HPW):
            h = hbase + m
            pltpu.sync_copy(tkc.at[j, h], rowbuf)

            def idx_src(c):
                return ikc.at[j, pl.ds(c * TCH, TCH)]

            def out_dst(c):
                return e_out.at[pl.ds(c * TCH, TCH), j, h]

            pltpu.async_copy(idx_src(0), idxbuf.at[0], si0)
            pltpu.async_copy(idx_src(1), idxbuf.at[1], si1)

            def t_body(g, carry):
                c0 = 2 * g
                c1 = c0 + 1
                pltpu.make_async_copy(idx_src(c0), idxbuf.at[0], si0).wait()

                @pl.when(g > 0)
                def _():
                    pltpu.make_async_copy(outbuf.at[0], out_dst(c0 - 2), so0).wait()

                gather_rows(0, TCH)
                pltpu.async_copy(outbuf.at[0], out_dst(c0), so0)

                @pl.when(c0 + 2 < NTCH)
                def _():
                    pltpu.async_copy(idx_src(c0 + 2), idxbuf.at[0], si0)

                pltpu.make_async_copy(idx_src(c1), idxbuf.at[1], si1).wait()

                @pl.when(g > 0)
                def _():
                    pltpu.make_async_copy(outbuf.at[1], out_dst(c1 - 2), so1).wait()

                gather_rows(1, TCH)
                pltpu.async_copy(outbuf.at[1], out_dst(c1), so1)

                @pl.when(c1 + 2 < NTCH)
                def _():
                    pltpu.async_copy(idx_src(c1 + 2), idxbuf.at[1], si1)

                return carry

            lax.fori_loop(0, NTCH // 2, t_body, 0)
            pltpu.make_async_copy(outbuf.at[0], out_dst(NTCH - 2), so0).wait()
            pltpu.make_async_copy(outbuf.at[1], out_dst(NTCH - 1), so1).wait()

        for m in range(SPW):
            p = wid * SPW + m
            kk = p // H
            h = p % H
            pltpu.sync_copy(tst.at[kk, h], rowbuf)
            pltpu.sync_copy(ist.at[kk], idxbuf.at[0, 0])
            gather_rows(0, 1)
            pltpu.sync_copy(outbuf.at[0, 0], st_out.at[kk, h])

    return k(tkc_t, kc_j, tst_t, st_k)


def _tc_assemble(xr_t, obs_t, e, W_kr, b_kr, W_obs, b_obs):
    def body(x_ref, o_ref, e_ref, wk, bk, wo, bo, kn_out, ob_out):
        wkv = wk[...][None, :, :, None]
        bkv = bk[...][None, :, :, None]
        x = x_ref[...]
        kn_out[:, 0:N_KR] = x[:, :, None, :] * wkv + bkv
        kn_out[:, N_KR:NK] = e_ref[...]
        o = o_ref[...]
        ob_out[...] = (o[:, :, None, :] * wo[...][None, :, :, None]
                       + bo[...][None, :, :, None])

    TB = 2
    return pl.pallas_call(
        body,
        grid=(T // TB,),
        in_specs=[
            pl.BlockSpec((TB, N_KR, B), lambda i: (i, 0, 0)),
            pl.BlockSpec((TB, N_OBS, B), lambda i: (i, 0, 0)),
            pl.BlockSpec((TB, N_KC, H, B), lambda i: (i, 0, 0, 0)),
            pl.BlockSpec((N_KR, H), lambda i: (0, 0)),
            pl.BlockSpec((N_KR, H), lambda i: (0, 0)),
            pl.BlockSpec((N_OBS, H), lambda i: (0, 0)),
            pl.BlockSpec((N_OBS, H), lambda i: (0, 0)),
        ],
        out_specs=[
            pl.BlockSpec((TB, NK, H, B), lambda i: (i, 0, 0, 0)),
            pl.BlockSpec((TB, N_OBS, H, B), lambda i: (i, 0, 0, 0)),
        ],
        out_shape=[
            jax.ShapeDtypeStruct((T, NK, H, B), jnp.float32),
            jax.ShapeDtypeStruct((T, N_OBS, H, B), jnp.float32),
        ],
    )(xr_t, obs_t, e, W_kr, b_kr, W_obs, b_obs)


def kernel(static, known_real, known_categorical, observed, E_static, E_kc,
           W_kr, b_kr, W_obs, b_obs):
    tkc_t = jnp.transpose(E_kc, (0, 2, 1))
    tst_t = jnp.transpose(E_static, (0, 2, 1))
    kc_j = jnp.transpose(known_categorical.astype(jnp.int32), (2, 1, 0))
    st_k = jnp.transpose(static.astype(jnp.int32), (1, 0))
    xr_t = jnp.transpose(known_real, (1, 2, 0))
    obs_t = jnp.transpose(observed, (1, 2, 0))

    e, st_g = _sc_gather(tkc_t, kc_j, tst_t, st_k)

    kn_t, ob_t = _tc_assemble(xr_t, obs_t, e, W_kr, b_kr, W_obs, b_obs)

    static_emb = jnp.transpose(st_g, (2, 0, 1))
    known = jnp.transpose(kn_t, (3, 0, 2, 1))
    obs = jnp.transpose(ob_t, (3, 0, 2, 1))
    return (static_emb, known, obs)


# speedup vs baseline: 6.8792x; 1.3959x over previous
import functools

import jax
import jax.numpy as jnp
from jax import lax
from jax.experimental import pallas as pl
from jax.experimental.pallas import tpu as pltpu
from jax.experimental.pallas import tpu_sc as plsc

B, T, H, V = 1024, 200, 64, 100000
N_STATIC, N_KC, N_KR, N_OBS = 3, 2, 4, 3
NK = N_KR + N_KC

NW = 32
HPW = (N_KC * H) // NW
SPW = (N_STATIC * H) // NW
TCH = 4
NTCH = T // TCH


def _sc_gather(tkc_t, kc_j, tst_t, st_k):
    mesh = plsc.VectorSubcoreMesh(core_axis_name="c", subcore_axis_name="s")

    @functools.partial(
        pl.kernel,
        mesh=mesh,
        out_type=(
            jax.ShapeDtypeStruct((T, N_KC, H, B), jnp.float32),
            jax.ShapeDtypeStruct((N_STATIC, H, B), jnp.float32),
        ),
        scratch_types=[
            pltpu.VMEM((V,), jnp.float32),
            pltpu.VMEM((2, TCH, B), jnp.int32),
            pltpu.VMEM((2, TCH, B), jnp.float32),
            pltpu.SemaphoreType.DMA,
            pltpu.SemaphoreType.DMA,
            pltpu.SemaphoreType.DMA,
            pltpu.SemaphoreType.DMA,
        ],
        compiler_params=pltpu.CompilerParams(
            use_tc_tiling_on_sc=True, needs_layout_passes=False),
    )
    def k(tkc, ikc, tst, ist, e_out, st_out, rowbuf, idxbuf, outbuf,
          si0, si1, so0, so1):
        wid = lax.axis_index("s") * 2 + lax.axis_index("c")

        def gather_rows(b, nrows):
            @plsc.parallel_loop(0, B, 16, unroll=8)
            def _(col):
                for r in range(nrows):
                    iv = idxbuf[b, r, pl.ds(col, 16)]
                    outbuf[b, r, pl.ds(col, 16)] = plsc.load_gather(rowbuf, [iv])

        j = wid // 16
        hbase = 4 * (wid % 16)
        for m in range(HPW):
            h = hbase + m
            pltpu.sync_copy(tkc.at[j, h], rowbuf)

            def idx_src(c):
                return ikc.at[j, pl.ds(c * TCH, TCH)]

            def out_dst(c):
                return e_out.at[pl.ds(c * TCH, TCH), j, h]

            pltpu.async_copy(idx_src(0), idxbuf.at[0], si0)
            pltpu.async_copy(idx_src(1), idxbuf.at[1], si1)

            def t_body(g, carry):
                c0 = 2 * g
                c1 = c0 + 1
                pltpu.make_async_copy(idx_src(c0), idxbuf.at[0], si0).wait()

                @pl.when(g > 0)
                def _():
                    pltpu.make_async_copy(outbuf.at[0], out_dst(c0 - 2), so0).wait()

                gather_rows(0, TCH)
                pltpu.async_copy(outbuf.at[0], out_dst(c0), so0)

                @pl.when(c0 + 2 < NTCH)
                def _():
                    pltpu.async_copy(idx_src(c0 + 2), idxbuf.at[0], si0)

                pltpu.make_async_copy(idx_src(c1), idxbuf.at[1], si1).wait()

                @pl.when(g > 0)
                def _():
                    pltpu.make_async_copy(outbuf.at[1], out_dst(c1 - 2), so1).wait()

                gather_rows(1, TCH)
                pltpu.async_copy(outbuf.at[1], out_dst(c1), so1)

                @pl.when(c1 + 2 < NTCH)
                def _():
                    pltpu.async_copy(idx_src(c1 + 2), idxbuf.at[1], si1)

                return carry

            lax.fori_loop(0, NTCH // 2, t_body, 0)
            pltpu.make_async_copy(outbuf.at[0], out_dst(NTCH - 2), so0).wait()
            pltpu.make_async_copy(outbuf.at[1], out_dst(NTCH - 1), so1).wait()

        for m in range(SPW):
            p = wid * SPW + m
            kk = p // H
            h = p % H
            pltpu.sync_copy(tst.at[kk, h], rowbuf)
            pltpu.sync_copy(ist.at[kk], idxbuf.at[0, 0])
            gather_rows(0, 1)
            pltpu.sync_copy(outbuf.at[0, 0], st_out.at[kk, h])

    return k(tkc_t, kc_j, tst_t, st_k)


def _tc_assemble(xr_t, obs_t, e, W_kr, b_kr, W_obs, b_obs):
    def body(x_ref, o_ref, e_ref, wk, bk, wo, bo, kn_out, ob_out):
        wkv = wk[...][None, :, :, None]
        bkv = bk[...][None, :, :, None]
        x = x_ref[...]
        kn_out[:, 0:N_KR] = x[:, :, None, :] * wkv + bkv
        kn_out[:, N_KR:NK] = e_ref[...]
        o = o_ref[...]
        ob_out[...] = (o[:, :, None, :] * wo[...][None, :, :, None]
                       + bo[...][None, :, :, None])

    TB = 2
    return pl.pallas_call(
        body,
        grid=(T // TB,),
        in_specs=[
            pl.BlockSpec((TB, N_KR, B), lambda i: (i, 0, 0)),
            pl.BlockSpec((TB, N_OBS, B), lambda i: (i, 0, 0)),
            pl.BlockSpec((TB, N_KC, H, B), lambda i: (i, 0, 0, 0)),
            pl.BlockSpec((N_KR, H), lambda i: (0, 0)),
            pl.BlockSpec((N_KR, H), lambda i: (0, 0)),
            pl.BlockSpec((N_OBS, H), lambda i: (0, 0)),
            pl.BlockSpec((N_OBS, H), lambda i: (0, 0)),
        ],
        out_specs=[
            pl.BlockSpec((TB, NK, H, B), lambda i: (i, 0, 0, 0)),
            pl.BlockSpec((TB, N_OBS, H, B), lambda i: (i, 0, 0, 0)),
        ],
        out_shape=[
            jax.ShapeDtypeStruct((T, NK, H, B), jnp.float32),
            jax.ShapeDtypeStruct((T, N_OBS, H, B), jnp.float32),
        ],
    )(xr_t, obs_t, e, W_kr, b_kr, W_obs, b_obs)


def kernel(static, known_real, known_categorical, observed, E_static, E_kc,
           W_kr, b_kr, W_obs, b_obs):
    tkc_t = jnp.transpose(E_kc, (0, 2, 1))
    tst_t = jnp.transpose(E_static, (0, 2, 1))
    kc_j = jnp.transpose(known_categorical.astype(jnp.int32), (2, 1, 0))
    st_k = jnp.transpose(static.astype(jnp.int32), (1, 0))
    xr_t = jnp.transpose(known_real, (1, 2, 0))
    obs_t = jnp.transpose(observed, (1, 2, 0))

    e, st_g = _sc_gather(tkc_t, kc_j, tst_t, st_k)

    kn_t, ob_t = _tc_assemble(xr_t, obs_t, e, W_kr, b_kr, W_obs, b_obs)

    static_emb = jnp.transpose(st_g, (2, 0, 1))
    known = jnp.transpose(kn_t, (3, 0, 2, 1))
    obs = jnp.transpose(ob_t, (3, 0, 2, 1))
    return (static_emb, known, obs)


# speedup vs baseline: 6.8861x; 1.0010x over previous
import functools

import jax
import jax.numpy as jnp
from jax import lax
from jax.experimental import pallas as pl
from jax.experimental.pallas import tpu as pltpu
from jax.experimental.pallas import tpu_sc as plsc

B, T, H, V = 1024, 200, 64, 100000
N_STATIC, N_KC, N_KR, N_OBS = 3, 2, 4, 3
NK = N_KR + N_KC

NW = 32
HPW = (N_KC * H) // NW
SPW = (N_STATIC * H) // NW
TCH = 4
NTCH = T // TCH


def _sc_gather(tkc_t, kc_j, tst_t, st_k):
    mesh = plsc.VectorSubcoreMesh(core_axis_name="c", subcore_axis_name="s")

    @functools.partial(
        pl.kernel,
        mesh=mesh,
        out_type=(
            jax.ShapeDtypeStruct((T, NK, H, B), jnp.float32),
            jax.ShapeDtypeStruct((N_STATIC, H, B), jnp.float32),
        ),
        scratch_types=[
            pltpu.VMEM((V,), jnp.float32),
            pltpu.VMEM((2, TCH, B), jnp.int32),
            pltpu.VMEM((2, TCH, B), jnp.float32),
            pltpu.SemaphoreType.DMA,
            pltpu.SemaphoreType.DMA,
            pltpu.SemaphoreType.DMA,
            pltpu.SemaphoreType.DMA,
        ],
        compiler_params=pltpu.CompilerParams(
            use_tc_tiling_on_sc=True, needs_layout_passes=False),
    )
    def k(tkc, ikc, tst, ist, e_out, st_out, rowbuf, idxbuf, outbuf,
          si0, si1, so0, so1):
        wid = lax.axis_index("s") * 2 + lax.axis_index("c")

        def gather_rows(b, nrows):
            @plsc.parallel_loop(0, B, 16, unroll=8)
            def _(col):
                for r in range(nrows):
                    iv = idxbuf[b, r, pl.ds(col, 16)]
                    outbuf[b, r, pl.ds(col, 16)] = plsc.load_gather(rowbuf, [iv])

        j = wid // 16
        hbase = 4 * (wid % 16)
        for m in range(HPW):
            h = hbase + m
            pltpu.sync_copy(tkc.at[j, h], rowbuf)

            def idx_src(c):
                return ikc.at[j, pl.ds(c * TCH, TCH)]

            def out_dst(c):
                return e_out.at[pl.ds(c * TCH, TCH), N_KR + j, h]

            pltpu.async_copy(idx_src(0), idxbuf.at[0], si0)
            pltpu.async_copy(idx_src(1), idxbuf.at[1], si1)

            def t_body(g, carry):
                c0 = 2 * g
                c1 = c0 + 1
                pltpu.make_async_copy(idx_src(c0), idxbuf.at[0], si0).wait()

                @pl.when(g > 0)
                def _():
                    pltpu.make_async_copy(outbuf.at[0], out_dst(c0 - 2), so0).wait()

                gather_rows(0, TCH)
                pltpu.async_copy(outbuf.at[0], out_dst(c0), so0)

                @pl.when(c0 + 2 < NTCH)
                def _():
                    pltpu.async_copy(idx_src(c0 + 2), idxbuf.at[0], si0)

                pltpu.make_async_copy(idx_src(c1), idxbuf.at[1], si1).wait()

                @pl.when(g > 0)
                def _():
                    pltpu.make_async_copy(outbuf.at[1], out_dst(c1 - 2), so1).wait()

                gather_rows(1, TCH)
                pltpu.async_copy(outbuf.at[1], out_dst(c1), so1)

                @pl.when(c1 + 2 < NTCH)
                def _():
                    pltpu.async_copy(idx_src(c1 + 2), idxbuf.at[1], si1)

                return carry

            lax.fori_loop(0, NTCH // 2, t_body, 0)
            pltpu.make_async_copy(outbuf.at[0], out_dst(NTCH - 2), so0).wait()
            pltpu.make_async_copy(outbuf.at[1], out_dst(NTCH - 1), so1).wait()

        for m in range(SPW):
            p = wid * SPW + m
            kk = p // H
            h = p % H
            pltpu.sync_copy(tst.at[kk, h], rowbuf)
            pltpu.sync_copy(ist.at[kk], idxbuf.at[0, 0])
            gather_rows(0, 1)
            pltpu.sync_copy(outbuf.at[0, 0], st_out.at[kk, h])

    return k(tkc_t, kc_j, tst_t, st_k)


def _tc_assemble(xr_t, obs_t, kn_partial, W_kr, b_kr, W_obs, b_obs):
    def body(x_ref, o_ref, kn_in, wk, bk, wo, bo, kn_out, ob_out):
        del kn_in
        wkv = wk[...][:, :, :, None]
        bkv = bk[...][:, :, :, None]
        x = x_ref[:, 0]
        kn_out[...] = x[:, :, None, :] * wkv + bkv
        o = o_ref[...]
        ob_out[...] = (o[:, :, None, :] * wo[...][None, :, :, None]
                       + bo[...][None, :, :, None])

    TB = 2
    return pl.pallas_call(
        body,
        grid=(T // TB, N_KR // 2),
        in_specs=[
            pl.BlockSpec((TB, 1, 2, B), lambda i, f: (i, f, 0, 0)),
            pl.BlockSpec((TB, N_OBS, B), lambda i, f: (i, 0, 0)),
            pl.BlockSpec((1, 1, H, B), lambda i, f: (0, 0, 0, 0)),
            pl.BlockSpec((1, 2, H), lambda i, f: (f, 0, 0)),
            pl.BlockSpec((1, 2, H), lambda i, f: (f, 0, 0)),
            pl.BlockSpec((N_OBS, H), lambda i, f: (0, 0)),
            pl.BlockSpec((N_OBS, H), lambda i, f: (0, 0)),
        ],
        out_specs=[
            pl.BlockSpec((TB, 2, H, B), lambda i, f: (i, f, 0, 0)),
            pl.BlockSpec((TB, N_OBS, H, B), lambda i, f: (i, 0, 0, 0)),
        ],
        out_shape=[
            jax.ShapeDtypeStruct((T, NK, H, B), jnp.float32),
            jax.ShapeDtypeStruct((T, N_OBS, H, B), jnp.float32),
        ],
        input_output_aliases={2: 0},
    )(xr_t.reshape(T, 2, 2, B), obs_t, kn_partial,
      W_kr.reshape(2, 2, H), b_kr.reshape(2, 2, H), W_obs, b_obs)


def kernel(static, known_real, known_categorical, observed, E_static, E_kc,
           W_kr, b_kr, W_obs, b_obs):
    tkc_t = jnp.transpose(E_kc, (0, 2, 1))
    tst_t = jnp.transpose(E_static, (0, 2, 1))
    kc_j = jnp.transpose(known_categorical.astype(jnp.int32), (2, 1, 0))
    st_k = jnp.transpose(static.astype(jnp.int32), (1, 0))
    xr_t = jnp.transpose(known_real, (1, 2, 0))
    obs_t = jnp.transpose(observed, (1, 2, 0))

    kn_partial, st_g = _sc_gather(tkc_t, kc_j, tst_t, st_k)

    kn_t, ob_t = _tc_assemble(xr_t, obs_t, kn_partial, W_kr, b_kr, W_obs, b_obs)

    static_emb = jnp.transpose(st_g, (2, 0, 1))
    known = jnp.transpose(kn_t, (3, 0, 2, 1))
    obs = jnp.transpose(ob_t, (3, 0, 2, 1))
    return (static_emb, known, obs)


# speedup vs baseline: 7.9423x; 1.1534x over previous
import functools

import jax
import jax.numpy as jnp
from jax import lax
from jax.experimental import pallas as pl
from jax.experimental.pallas import tpu as pltpu
from jax.experimental.pallas import tpu_sc as plsc

B, T, H, V = 1024, 200, 64, 100000
N_STATIC, N_KC, N_KR, N_OBS = 3, 2, 4, 3
NK = N_KR + N_KC

NW = 32
HPW = (N_KC * H) // NW
SPW = (N_STATIC * H) // NW
TCH = 4
NTCH = T // TCH


def _sc_gather(tkc_t, kc_j, tst_t, st_k):
    mesh = plsc.VectorSubcoreMesh(core_axis_name="c", subcore_axis_name="s")

    @functools.partial(
        pl.kernel,
        mesh=mesh,
        out_type=(
            jax.ShapeDtypeStruct((T, NK, H, B), jnp.float32),
            jax.ShapeDtypeStruct((N_STATIC, H, B), jnp.float32),
        ),
        scratch_types=[
            pltpu.VMEM((V,), jnp.float32),
            pltpu.VMEM((2, TCH, B), jnp.int32),
            pltpu.VMEM((2, TCH, B), jnp.float32),
            pltpu.SemaphoreType.DMA,
            pltpu.SemaphoreType.DMA,
            pltpu.SemaphoreType.DMA,
            pltpu.SemaphoreType.DMA,
        ],
        compiler_params=pltpu.CompilerParams(
            use_tc_tiling_on_sc=True, needs_layout_passes=False),
    )
    def k(tkc, ikc, tst, ist, e_out, st_out, rowbuf, idxbuf, outbuf,
          si0, si1, so0, so1):
        wid = lax.axis_index("s") * 2 + lax.axis_index("c")

        def gather_rows(b, nrows):
            @plsc.parallel_loop(0, B, 16, unroll=8)
            def _(col):
                for r in range(nrows):
                    iv = idxbuf[b, r, pl.ds(col, 16)]
                    outbuf[b, r, pl.ds(col, 16)] = plsc.load_gather(rowbuf, [iv])

        j = wid // 16
        hbase = 4 * (wid % 16)
        for m in range(HPW):
            h = hbase + m
            pltpu.sync_copy(tkc.at[j, h], rowbuf)

            def idx_src(c):
                return ikc.at[j, pl.ds(c * TCH, TCH)]

            def out_dst(c):
                return e_out.at[pl.ds(c * TCH, TCH), N_KR + j, h]

            pltpu.async_copy(idx_src(0), idxbuf.at[0], si0)
            pltpu.async_copy(idx_src(1), idxbuf.at[1], si1)

            def t_body(g, carry):
                c0 = 2 * g
                c1 = c0 + 1
                pltpu.make_async_copy(idx_src(c0), idxbuf.at[0], si0).wait()

                @pl.when(g > 0)
                def _():
                    pltpu.make_async_copy(outbuf.at[0], out_dst(c0 - 2), so0).wait()

                gather_rows(0, TCH)
                pltpu.async_copy(outbuf.at[0], out_dst(c0), so0)

                @pl.when(c0 + 2 < NTCH)
                def _():
                    pltpu.async_copy(idx_src(c0 + 2), idxbuf.at[0], si0)

                pltpu.make_async_copy(idx_src(c1), idxbuf.at[1], si1).wait()

                @pl.when(g > 0)
                def _():
                    pltpu.make_async_copy(outbuf.at[1], out_dst(c1 - 2), so1).wait()

                gather_rows(1, TCH)
                pltpu.async_copy(outbuf.at[1], out_dst(c1), so1)

                @pl.when(c1 + 2 < NTCH)
                def _():
                    pltpu.async_copy(idx_src(c1 + 2), idxbuf.at[1], si1)

                return carry

            lax.fori_loop(0, NTCH // 2, t_body, 0)
            pltpu.make_async_copy(outbuf.at[0], out_dst(NTCH - 2), so0).wait()
            pltpu.make_async_copy(outbuf.at[1], out_dst(NTCH - 1), so1).wait()

        for m in range(SPW):
            p = wid * SPW + m
            kk = p // H
            h = p % H
            pltpu.sync_copy(tst.at[kk, h], rowbuf)
            pltpu.sync_copy(ist.at[kk], idxbuf.at[0, 0])
            gather_rows(0, 1)
            pltpu.sync_copy(outbuf.at[0, 0], st_out.at[kk, h])

    return k(tkc_t, kc_j, tst_t, st_k)


def _tc_assemble(xr_t, obs_t, kn_partial, W_kr, b_kr, W_obs, b_obs):
    TB = 4

    def body(x_ref, kn_in, wk, bk, kn_out):
        del kn_in
        wkv = wk[...][:, :, :, None]
        bkv = bk[...][:, :, :, None]
        x = x_ref[:, 0]
        kn_out[...] = x[:, :, None, :] * wkv + bkv

    kn_t = pl.pallas_call(
        body,
        grid=(T // TB, N_KR // 2),
        in_specs=[
            pl.BlockSpec((TB, 1, 2, B), lambda i, f: (i, f, 0, 0)),
            pl.BlockSpec((1, 1, H, B), lambda i, f: (0, 0, 0, 0)),
            pl.BlockSpec((1, 2, H), lambda i, f: (f, 0, 0)),
            pl.BlockSpec((1, 2, H), lambda i, f: (f, 0, 0)),
        ],
        out_specs=[
            pl.BlockSpec((TB, 2, H, B), lambda i, f: (i, f, 0, 0)),
        ],
        out_shape=[
            jax.ShapeDtypeStruct((T, NK, H, B), jnp.float32),
        ],
        input_output_aliases={1: 0},
    )(xr_t.reshape(T, 2, 2, B), kn_partial,
      W_kr.reshape(2, 2, H), b_kr.reshape(2, 2, H))[0]

    def obs_body(o_ref, wo, bo, ob_out):
        o = o_ref[...]
        ob_out[...] = (o[:, :, None, :] * wo[...][None, :, :, None]
                       + bo[...][None, :, :, None])

    ob_t = pl.pallas_call(
        obs_body,
        grid=(T // TB,),
        in_specs=[
            pl.BlockSpec((TB, N_OBS, B), lambda i: (i, 0, 0)),
            pl.BlockSpec((N_OBS, H), lambda i: (0, 0)),
            pl.BlockSpec((N_OBS, H), lambda i: (0, 0)),
        ],
        out_specs=[
            pl.BlockSpec((TB, N_OBS, H, B), lambda i: (i, 0, 0, 0)),
        ],
        out_shape=[
            jax.ShapeDtypeStruct((T, N_OBS, H, B), jnp.float32),
        ],
    )(obs_t, W_obs, b_obs)[0]
    return kn_t, ob_t


def kernel(static, known_real, known_categorical, observed, E_static, E_kc,
           W_kr, b_kr, W_obs, b_obs):
    tkc_t = jnp.transpose(E_kc, (0, 2, 1))
    tst_t = jnp.transpose(E_static, (0, 2, 1))
    kc_j = jnp.transpose(known_categorical.astype(jnp.int32), (2, 1, 0))
    st_k = jnp.transpose(static.astype(jnp.int32), (1, 0))
    xr_t = jnp.transpose(known_real, (1, 2, 0))
    obs_t = jnp.transpose(observed, (1, 2, 0))

    kn_partial, st_g = _sc_gather(tkc_t, kc_j, tst_t, st_k)

    kn_t, ob_t = _tc_assemble(xr_t, obs_t, kn_partial, W_kr, b_kr, W_obs, b_obs)

    static_emb = jnp.transpose(st_g, (2, 0, 1))
    known = jnp.transpose(kn_t, (3, 0, 2, 1))
    obs = jnp.transpose(ob_t, (3, 0, 2, 1))
    return (static_emb, known, obs)


# speedup vs baseline: 10.6525x; 1.3412x over previous
import functools

import jax
import jax.numpy as jnp
from jax import lax
from jax.experimental import pallas as pl
from jax.experimental.pallas import tpu as pltpu
from jax.experimental.pallas import tpu_sc as plsc

B, T, H, V = 1024, 200, 64, 100000
N_STATIC, N_KC, N_KR, N_OBS = 3, 2, 4, 3
NK = N_KR + N_KC

NW = 32
HPW = (N_KC * H) // NW
SPW = (N_STATIC * H) // NW
TCH = 4
NTCH = T // TCH


def _sc_gather(tkc_t, kc_j, tst_t, st_k):
    mesh = plsc.VectorSubcoreMesh(core_axis_name="c", subcore_axis_name="s")

    @functools.partial(
        pl.kernel,
        mesh=mesh,
        out_type=(
            jax.ShapeDtypeStruct((T, NK, H, B), jnp.float32),
            jax.ShapeDtypeStruct((N_STATIC, H, B), jnp.float32),
        ),
        scratch_types=[
            pltpu.VMEM((V,), jnp.float32),
            pltpu.VMEM((2, TCH, B), jnp.int32),
            pltpu.VMEM((2, TCH, B), jnp.float32),
            pltpu.VMEM_SHARED((T, B), jnp.int32),
            pltpu.SemaphoreType.DMA,
            pltpu.SemaphoreType.DMA,
            pltpu.SemaphoreType.DMA,
            pltpu.SemaphoreType.DMA,
        ],
        compiler_params=pltpu.CompilerParams(
            use_tc_tiling_on_sc=True, needs_layout_passes=False),
    )
    def k(tkc, ikc, tst, ist, e_out, st_out, rowbuf, idxbuf, outbuf, idx_sh,
          si0, si1, so0, so1):
        wid = lax.axis_index("s") * 2 + lax.axis_index("c")

        j = lax.axis_index("c")
        hbase = 4 * lax.axis_index("s")

        @pl.when(lax.axis_index("s") == 0)
        def _():
            pltpu.sync_copy(ikc.at[j], idx_sh)

        plsc.subcore_barrier()

        def gather_rows(b, nrows):
            @plsc.parallel_loop(0, B, 16, unroll=8)
            def _(col):
                for r in range(nrows):
                    iv = idxbuf[b, r, pl.ds(col, 16)]
                    outbuf[b, r, pl.ds(col, 16)] = plsc.load_gather(rowbuf, [iv])

        for m in range(HPW):
            h = hbase + m
            pltpu.sync_copy(tkc.at[j, h], rowbuf)

            def idx_src(c):
                return idx_sh.at[pl.ds(c * TCH, TCH)]

            def out_dst(c):
                return e_out.at[pl.ds(c * TCH, TCH), N_KR + j, h]

            pltpu.async_copy(idx_src(0), idxbuf.at[0], si0)
            pltpu.async_copy(idx_src(1), idxbuf.at[1], si1)

            def t_body(g, carry):
                c0 = 2 * g
                c1 = c0 + 1
                pltpu.make_async_copy(idx_src(c0), idxbuf.at[0], si0).wait()

                @pl.when(g > 0)
                def _():
                    pltpu.make_async_copy(outbuf.at[0], out_dst(c0 - 2), so0).wait()

                gather_rows(0, TCH)
                pltpu.async_copy(outbuf.at[0], out_dst(c0), so0)

                @pl.when(c0 + 2 < NTCH)
                def _():
                    pltpu.async_copy(idx_src(c0 + 2), idxbuf.at[0], si0)

                pltpu.make_async_copy(idx_src(c1), idxbuf.at[1], si1).wait()

                @pl.when(g > 0)
                def _():
                    pltpu.make_async_copy(outbuf.at[1], out_dst(c1 - 2), so1).wait()

                gather_rows(1, TCH)
                pltpu.async_copy(outbuf.at[1], out_dst(c1), so1)

                @pl.when(c1 + 2 < NTCH)
                def _():
                    pltpu.async_copy(idx_src(c1 + 2), idxbuf.at[1], si1)

                return carry

            lax.fori_loop(0, NTCH // 2, t_body, 0)
            pltpu.make_async_copy(outbuf.at[0], out_dst(NTCH - 2), so0).wait()
            pltpu.make_async_copy(outbuf.at[1], out_dst(NTCH - 1), so1).wait()

        for m in range(SPW):
            p = wid * SPW + m
            kk = p // H
            h = p % H
            pltpu.sync_copy(tst.at[kk, h], rowbuf)
            pltpu.sync_copy(ist.at[kk], idxbuf.at[0, 0])
            gather_rows(0, 1)
            pltpu.sync_copy(outbuf.at[0, 0], st_out.at[kk, h])

    return k(tkc_t, kc_j, tst_t, st_k)


def _tc_assemble(xr_t, obs_t, kn_partial, W_kr, b_kr, W_obs, b_obs):
    TB = 4

    def body(x_ref, kn_in, wk, bk, kn_out):
        del kn_in
        wkv = wk[...][:, :, :, None]
        bkv = bk[...][:, :, :, None]
        x = x_ref[:, 0]
        kn_out[...] = x[:, :, None, :] * wkv + bkv

    kn_t = pl.pallas_call(
        body,
        grid=(T // TB, N_KR // 2),
        in_specs=[
            pl.BlockSpec((TB, 1, 2, B), lambda i, f: (i, f, 0, 0)),
            pl.BlockSpec((1, 1, H, B), lambda i, f: (0, 0, 0, 0)),
            pl.BlockSpec((1, 2, H), lambda i, f: (f, 0, 0)),
            pl.BlockSpec((1, 2, H), lambda i, f: (f, 0, 0)),
        ],
        out_specs=[
            pl.BlockSpec((TB, 2, H, B), lambda i, f: (i, f, 0, 0)),
        ],
        out_shape=[
            jax.ShapeDtypeStruct((T, NK, H, B), jnp.float32),
        ],
        input_output_aliases={1: 0},
    )(xr_t.reshape(T, 2, 2, B), kn_partial,
      W_kr.reshape(2, 2, H), b_kr.reshape(2, 2, H))[0]

    def obs_body(o_ref, wo, bo, ob_out):
        o = o_ref[...]
        ob_out[...] = (o[:, :, None, :] * wo[...][None, :, :, None]
                       + bo[...][None, :, :, None])

    ob_t = pl.pallas_call(
        obs_body,
        grid=(T // TB,),
        in_specs=[
            pl.BlockSpec((TB, N_OBS, B), lambda i: (i, 0, 0)),
            pl.BlockSpec((N_OBS, H), lambda i: (0, 0)),
            pl.BlockSpec((N_OBS, H), lambda i: (0, 0)),
        ],
        out_specs=[
            pl.BlockSpec((TB, N_OBS, H, B), lambda i: (i, 0, 0, 0)),
        ],
        out_shape=[
            jax.ShapeDtypeStruct((T, N_OBS, H, B), jnp.float32),
        ],
    )(obs_t, W_obs, b_obs)[0]
    return kn_t, ob_t


def kernel(static, known_real, known_categorical, observed, E_static, E_kc,
           W_kr, b_kr, W_obs, b_obs):
    tkc_t = jnp.transpose(E_kc, (0, 2, 1))
    tst_t = jnp.transpose(E_static, (0, 2, 1))
    kc_j = jnp.transpose(known_categorical.astype(jnp.int32), (2, 1, 0))
    st_k = jnp.transpose(static.astype(jnp.int32), (1, 0))
    xr_t = jnp.transpose(known_real, (1, 2, 0))
    obs_t = jnp.transpose(observed, (1, 2, 0))

    kn_partial, st_g = _sc_gather(tkc_t, kc_j, tst_t, st_k)

    kn_t, ob_t = _tc_assemble(xr_t, obs_t, kn_partial, W_kr, b_kr, W_obs, b_obs)

    static_emb = jnp.transpose(st_g, (2, 0, 1))
    known = jnp.transpose(kn_t, (3, 0, 2, 1))
    obs = jnp.transpose(ob_t, (3, 0, 2, 1))
    return (static_emb, known, obs)
